# baseline (speedup 1.0000x reference)
import jax
import jax.numpy as jnp
from jax.experimental import pallas as pl
from jax.experimental.pallas import tpu as pltpu

BS = 1024
NC = 200
NCU = 50
ED = 128
T = 128


def _body(cs_ref, task_ref, dyn_ref, mf_ref, st_ref, sth_ref, mfs_ref,
          nd_ref, nm_ref, d_ref, dmf_ref, s_ref, sh_ref, mfso_ref):
    b = pl.program_id(0)
    cs = cs_ref[b]
    task = task_ref[b].astype(jnp.float32)

    lane = jax.lax.broadcasted_iota(jnp.int32, (1, NC), 1)
    onehot = (lane == cs)
    onehot_f = onehot.astype(jnp.float32)

    dyn = dyn_ref[0]
    mf = mf_ref[0]

    nd_ref[0, :T, :] = dyn
    nd_ref[0, T:T + 1, :] = jnp.where(onehot, task, 0.0)
    nm_ref[0, :T, :] = mf
    nm_ref[0, T:T + 1, :] = jnp.where(onehot, 0.0, -jnp.inf)

    d_ref[0, 0, :] = jnp.sum(dyn * onehot_f, axis=1)
    dmf_ref[0, 0, :] = jnp.sum(mf * onehot_f, axis=1)

    s_ref[0] = st_ref[0]
    sh_ref[0] = sth_ref[0]
    mfso_ref[0] = mfs_ref[0]


def kernel(static, static_h, dynamic, mask_f, mask_fs, couriers_selected,
           sensingtask_selected):
    bs = static.shape[0]
    cs_flat = couriers_selected[:, 0]
    task_flat = sensingtask_selected[:, 0]

    static_r = static.reshape(bs * NC, 1, 2 * NCU)
    static_h_r = static_h.reshape(bs * NC, 1, ED)
    mask_fs_r = mask_fs.reshape(bs * NC, 1, NCU)

    def at_b(i, cs_r, task_r):
        return (i, 0, 0)

    def at_cs(i, cs_r, task_r):
        return (i * NC + cs_r[i], 0, 0)

    grid_spec = pltpu.PrefetchScalarGridSpec(
        num_scalar_prefetch=2,
        grid=(bs,),
        in_specs=[
            pl.BlockSpec((1, T, NC), at_b),
            pl.BlockSpec((1, T, NC), at_b),
            pl.BlockSpec((1, 1, 2 * NCU), at_cs),
            pl.BlockSpec((1, 1, ED), at_cs),
            pl.BlockSpec((1, 1, NCU), at_cs),
        ],
        out_specs=[
            pl.BlockSpec((1, T + 1, NC), at_b),
            pl.BlockSpec((1, T + 1, NC), at_b),
            pl.BlockSpec((1, 1, T), at_b),
            pl.BlockSpec((1, 1, T), at_b),
            pl.BlockSpec((1, 1, 2 * NCU), at_b),
            pl.BlockSpec((1, 1, ED), at_b),
            pl.BlockSpec((1, 1, NCU), at_b),
        ],
    )

    out_shapes = [
        jax.ShapeDtypeStruct((bs, T + 1, NC), jnp.float32),
        jax.ShapeDtypeStruct((bs, T + 1, NC), jnp.float32),
        jax.ShapeDtypeStruct((bs, 1, T), jnp.float32),
        jax.ShapeDtypeStruct((bs, 1, T), jnp.float32),
        jax.ShapeDtypeStruct((bs, 1, 2 * NCU), jnp.float32),
        jax.ShapeDtypeStruct((bs, 1, ED), jnp.float32),
        jax.ShapeDtypeStruct((bs, 1, NCU), jnp.float32),
    ]

    nd, nm, d, mf, s, sh, mfs = pl.pallas_call(
        _body,
        grid_spec=grid_spec,
        out_shape=out_shapes,
    )(cs_flat, task_flat, dynamic, mask_f, static_r, static_h_r, mask_fs_r)

    return (s, sh, d.reshape(bs, T, 1), mf.reshape(bs, T, 1), mfs, nd, nm)


# speedup vs baseline: 1.2751x; 1.2751x over previous
import functools

import jax
import jax.numpy as jnp
from jax.experimental import pallas as pl
from jax.experimental.pallas import tpu as pltpu
from jax.experimental.pallas import tpu_sc as plsc

BS = 1024
NC = 200
NCU = 50
ED = 128
T = 128

G = 16
SC_WORKERS = 32
BPW = BS // SC_WORKERS


def _tc_body(cs_ref, task_ref, dyn_ref, mf_ref, nd_ref, nm_ref, d_ref, dmf_ref):
    i = pl.program_id(0)

    nd_ref[:, :T, :] = dyn_ref[...]
    nm_ref[:, :T, :] = mf_ref[...]

    lane = jax.lax.broadcasted_iota(jnp.int32, (1, NC), 1)
    sub = jax.lax.broadcasted_iota(jnp.int32, (NC, 1), 0)

    for g in range(G):
        b = i * G + g
        cs = cs_ref[b]
        task = task_ref[b].astype(jnp.float32)

        onehot_row = (lane == cs)
        nd_ref[g, T:T + 1, :] = jnp.where(onehot_row, task, 0.0)
        nm_ref[g, T:T + 1, :] = jnp.where(onehot_row, 0.0, -jnp.inf)

        onehot_col = (sub == cs).astype(jnp.float32)
        d_ref[g, :, :] = jax.lax.dot(
            dyn_ref[g], onehot_col,
            precision=jax.lax.Precision.HIGHEST,
            preferred_element_type=jnp.float32)
        dmf_ref[g, :, :] = jax.lax.dot(
            mf_ref[g], onehot_col,
            precision=jax.lax.Precision.HIGHEST,
            preferred_element_type=jnp.float32)


def _sc_gather_body(cs_hbm, st_hbm, sth_hbm, mfs_hbm, s_out, sh_out, mfs_out,
                    cs_v, idx_v, r_s, r_sh, r_mfs, sem, sem2):
    c = jax.lax.axis_index("c")
    s = jax.lax.axis_index("s")
    wid = s * 2 + c
    base = wid * BPW

    pltpu.sync_copy(cs_hbm.at[pl.ds(base, BPW)], cs_v)
    for j in range(BPW // 16):
        off = base + j * 16
        iota = jax.lax.broadcasted_iota(jnp.int32, (16,), 0)
        idx_v[pl.ds(j * 16, 16)] = cs_v[pl.ds(j * 16, 16)] + (iota + off) * NC

    sh_dma = pltpu.async_copy(sth_hbm.at[idx_v], r_sh, sem)

    fired = []
    for j16 in range(BPW // 16):
        vec = idx_v[pl.ds(j16 * 16, 16)]
        for l in range(16):
            j = j16 * 16 + l
            row = vec[l]
            fired.append(pltpu.async_copy(
                st_hbm.at[pl.ds(row, 1)], r_s.at[pl.ds(j, 1)], sem2))
            fired.append(pltpu.async_copy(
                mfs_hbm.at[pl.ds(row, 1)], r_mfs.at[pl.ds(j, 1)], sem2))
    sh_dma.wait()
    for dma in fired:
        dma.wait()

    pltpu.sync_copy(r_s, s_out.at[pl.ds(base, BPW)])
    pltpu.sync_copy(r_sh, sh_out.at[pl.ds(base, BPW)])
    pltpu.sync_copy(r_mfs, mfs_out.at[pl.ds(base, BPW)])


def kernel(static, static_h, dynamic, mask_f, mask_fs, couriers_selected,
           sensingtask_selected):
    bs = static.shape[0]
    cs_flat = couriers_selected[:, 0]
    task_flat = sensingtask_selected[:, 0]

    def at_b(i, cs_r, task_r):
        return (i, 0, 0)

    grid_spec = pltpu.PrefetchScalarGridSpec(
        num_scalar_prefetch=2,
        grid=(bs // G,),
        in_specs=[
            pl.BlockSpec((G, T, NC), at_b),
            pl.BlockSpec((G, T, NC), at_b),
        ],
        out_specs=[
            pl.BlockSpec((G, T + 1, NC), at_b),
            pl.BlockSpec((G, T + 1, NC), at_b),
            pl.BlockSpec((G, T, 1), at_b),
            pl.BlockSpec((G, T, 1), at_b),
        ],
    )

    nd, nm, d, mf = pl.pallas_call(
        _tc_body,
        grid_spec=grid_spec,
        out_shape=[
            jax.ShapeDtypeStruct((bs, T + 1, NC), jnp.float32),
            jax.ShapeDtypeStruct((bs, T + 1, NC), jnp.float32),
            jax.ShapeDtypeStruct((bs, T, 1), jnp.float32),
            jax.ShapeDtypeStruct((bs, T, 1), jnp.float32),
        ],
    )(cs_flat, task_flat, dynamic, mask_f)

    sc_call = pl.kernel(
        _sc_gather_body,
        out_type=[
            jax.ShapeDtypeStruct((bs, 2 * NCU), jnp.float32),
            jax.ShapeDtypeStruct((bs, ED), jnp.float32),
            jax.ShapeDtypeStruct((bs, NCU), jnp.float32),
        ],
        mesh=plsc.VectorSubcoreMesh(core_axis_name="c", subcore_axis_name="s",
                                    num_cores=2, num_subcores=16),
        scratch_types=[
            pltpu.VMEM((BPW,), jnp.int32),
            pltpu.VMEM((BPW,), jnp.int32),
            pltpu.VMEM((BPW, 2 * NCU), jnp.float32),
            pltpu.VMEM((BPW, ED), jnp.float32),
            pltpu.VMEM((BPW, NCU), jnp.float32),
            pltpu.SemaphoreType.DMA,
            pltpu.SemaphoreType.DMA,
        ],
    )
    s_f, sh_f, mfs_f = sc_call(
        cs_flat,
        static.reshape(bs * NC, 2 * NCU),
        static_h.reshape(bs * NC, ED),
        mask_fs.reshape(bs * NC, NCU),
    )

    return (s_f[:, None, :], sh_f[:, None, :], d, mf, mfs_f[:, None, :], nd, nm)
